# baseline (speedup 1.0000x reference)
import functools

import jax
import jax.numpy as jnp
from jax import lax
from jax.experimental import pallas as pl
from jax.experimental.pallas import tpu as pltpu
from jax.experimental.pallas import tpu_sc as plsc

EPS = 1e-05
NC, NS, L = 2, 16, 16
NW = NC * NS
BG = 1024
NPAD = 10240

BLK = 128


def _rowsums_body(h_ref, s_ref, q_ref):
    hb = h_ref[...]
    s_ref[...] = jnp.sum(hb, axis=1)
    q_ref[...] = jnp.sum(hb * hb, axis=1)


def _norm_body(h_ref, ps_ref, pq_ref, pd_ref, gamma_ref, beta_ref, out_ref):
    hb = h_ref[...]
    d = hb.shape[1]
    ones_m = jnp.ones((8, NW), jnp.float32)
    nb_s = jax.lax.dot(ones_m, ps_ref[...],
                       preferred_element_type=jnp.float32)[0]
    nb_q = jax.lax.dot(ones_m, pq_ref[...],
                       preferred_element_type=jnp.float32)[0]
    deg = jax.lax.dot(ones_m, pd_ref[...],
                      preferred_element_type=jnp.float32)[0]
    total_sum = nb_s + jnp.sum(hb, axis=1)
    total_sq = nb_q + jnp.sum(hb * hb, axis=1)
    n_elem = (deg + 1.0) * d
    mean = total_sum / n_elem
    var = (total_sq - n_elem * mean * mean) / (n_elem - 1.0)
    std = jnp.sqrt(jnp.maximum(var, 0.0))
    has_msg = deg > 0
    mean = jnp.where(has_msg, mean, 0.0)
    std = jnp.where(has_msg, std, 0.0)
    norm_h = (hb - mean[:, None]) / (std[:, None] + EPS)
    out_ref[...] = gamma_ref[...] * norm_h + beta_ref[...]


def _make_sc_agg(n_nodes, n_edges):
    nblk = n_edges // BLK
    blk_lo = nblk // NW
    n_hi = nblk - blk_lo * NW
    nw_lo = NW - n_hi
    e_lo = blk_lo * BLK
    e_hi = e_lo + BLK

    mesh = plsc.VectorSubcoreMesh(core_axis_name="c", subcore_axis_name="s")
    fdt = jax.ShapeDtypeStruct((NW, NPAD), jnp.float32)

    @functools.partial(
        pl.kernel,
        out_type=(fdt, fdt, fdt),
        mesh=mesh,
        compiler_params=pltpu.CompilerParams(needs_layout_passes=False),
        scratch_types=[
            pltpu.VMEM((n_nodes,), jnp.float32),
            pltpu.VMEM((n_nodes,), jnp.float32),
            pltpu.VMEM((NPAD,), jnp.float32),
            pltpu.VMEM((NPAD,), jnp.float32),
            pltpu.VMEM((NPAD,), jnp.float32),
            pltpu.VMEM((2, e_hi), jnp.int32),
            pltpu.SemaphoreType.DMA,
            pltpu.SemaphoreType.DMA,
            pltpu.SemaphoreType.DMA,
        ],
    )
    def sc_agg(s_hbm, q_hbm, ei_hbm, os_hbm, oq_hbm, od_hbm,
               s_v, q_v, acc_s, acc_q, acc_d, ei_v,
               sem0, sem1, sem2):
        wid = lax.axis_index("c") * NS + lax.axis_index("s")
        is_hi = wid >= nw_lo
        base = jnp.where(is_hi,
                         nw_lo * e_lo + (wid - nw_lo) * e_hi,
                         wid * e_lo)
        cp0 = pltpu.async_copy(s_hbm, s_v, sem0)
        cp1 = pltpu.async_copy(q_hbm, q_v, sem1)
        cp2 = pltpu.async_copy(ei_hbm.at[:, pl.ds(base, e_lo)],
                               ei_v.at[:, pl.ds(0, e_lo)], sem2)

        zeros = jnp.zeros((L,), jnp.float32)

        @plsc.parallel_loop(0, NPAD // L, step=1, unroll=2)
        def zero_body(i):
            b = i * L
            acc_s[pl.ds(b, L)] = zeros
            acc_q[pl.ds(b, L)] = zeros
            acc_d[pl.ds(b, L)] = zeros

        cp0.wait()
        cp1.wait()
        cp2.wait()

        @pl.when(is_hi)
        def _tail_dma():
            pltpu.sync_copy(ei_hbm.at[:, pl.ds(base + e_lo, BLK)],
                            ei_v.at[:, pl.ds(e_lo, BLK)])

        ones = jnp.ones((L,), jnp.float32)

        def _process(b):
            si = ei_v[0, pl.ds(b, L)]
            di = ei_v[1, pl.ds(b, L)]
            sv = plsc.load_gather(s_v, [si])
            qv = plsc.load_gather(q_v, [si])
            plsc.addupdate_scatter(acc_s, [di], sv)
            plsc.addupdate_scatter(acc_q, [di], qv)
            plsc.addupdate_scatter(acc_d, [di], ones)

        @plsc.parallel_loop(0, e_lo, step=L, unroll=4)
        def edge_body(b):
            _process(b)

        @pl.when(is_hi)
        def _tail_edges():
            @plsc.parallel_loop(e_lo, e_hi, step=L, unroll=1)
            def tail_body(b):
                _process(b)

        pltpu.sync_copy(acc_s, os_hbm.at[wid])
        pltpu.sync_copy(acc_q, oq_hbm.at[wid])
        pltpu.sync_copy(acc_d, od_hbm.at[wid])

    return sc_agg


def kernel(h, edge_index, gamma, beta):
    n, d = h.shape
    e = edge_index.shape[1]

    g = pl.cdiv(n, BG)
    s, q = pl.pallas_call(
        _rowsums_body,
        grid=(g,),
        in_specs=[pl.BlockSpec((BG, d), lambda i: (i, 0))],
        out_specs=(
            pl.BlockSpec((BG,), lambda i: (i,)),
            pl.BlockSpec((BG,), lambda i: (i,)),
        ),
        out_shape=(
            jax.ShapeDtypeStruct((n,), jnp.float32),
            jax.ShapeDtypeStruct((n,), jnp.float32),
        ),
    )(h)

    ps, pq, pd = _make_sc_agg(n, e)(s, q, edge_index)

    part_spec = pl.BlockSpec((NW, BG), lambda i: (0, i))
    vec_spec = pl.BlockSpec((d,), lambda i: (0,))
    out = pl.pallas_call(
        _norm_body,
        grid=(g,),
        in_specs=[
            pl.BlockSpec((BG, d), lambda i: (i, 0)),
            part_spec, part_spec, part_spec,
            vec_spec, vec_spec,
        ],
        out_specs=pl.BlockSpec((BG, d), lambda i: (i, 0)),
        out_shape=jax.ShapeDtypeStruct((n, d), h.dtype),
    )(h, ps, pq, pd, gamma, beta)
    return out


# speedup vs baseline: 1.0749x; 1.0749x over previous
import functools

import jax
import jax.numpy as jnp
from jax import lax
from jax.experimental import pallas as pl
from jax.experimental.pallas import tpu as pltpu
from jax.experimental.pallas import tpu_sc as plsc

EPS = 1e-05
NC, NS, L = 2, 16, 16
NW = NC * NS
BG = 5120
NPAD = 10240

BLK = 128


def _rowsums_body(h_ref, s_ref, q_ref):
    hb = h_ref[...]
    s_ref[...] = jnp.sum(hb, axis=1)
    q_ref[...] = jnp.sum(hb * hb, axis=1)


def _norm_body(h_ref, ps_ref, pq_ref, pd_ref, gamma_ref, beta_ref, out_ref):
    hb = h_ref[...]
    d = hb.shape[1]
    ones_m = jnp.ones((8, NW), jnp.float32)
    nb_s = jax.lax.dot(ones_m, ps_ref[...],
                       preferred_element_type=jnp.float32)[0]
    nb_q = jax.lax.dot(ones_m, pq_ref[...],
                       preferred_element_type=jnp.float32)[0]
    deg = jax.lax.dot(ones_m, pd_ref[...],
                      preferred_element_type=jnp.float32)[0]
    total_sum = nb_s + jnp.sum(hb, axis=1)
    total_sq = nb_q + jnp.sum(hb * hb, axis=1)
    n_elem = (deg + 1.0) * d
    mean = total_sum / n_elem
    var = (total_sq - n_elem * mean * mean) / (n_elem - 1.0)
    std = jnp.sqrt(jnp.maximum(var, 0.0))
    has_msg = deg > 0
    mean = jnp.where(has_msg, mean, 0.0)
    std = jnp.where(has_msg, std, 0.0)
    norm_h = (hb - mean[:, None]) / (std[:, None] + EPS)
    out_ref[...] = gamma_ref[...] * norm_h + beta_ref[...]


def _make_sc_agg(n_nodes, n_edges):
    nblk = n_edges // BLK
    blk_lo = nblk // NW
    n_hi = nblk - blk_lo * NW
    nw_lo = NW - n_hi
    e_lo = blk_lo * BLK
    e_hi = e_lo + BLK

    mesh = plsc.VectorSubcoreMesh(core_axis_name="c", subcore_axis_name="s")
    fdt = jax.ShapeDtypeStruct((NW, NPAD), jnp.float32)

    @functools.partial(
        pl.kernel,
        out_type=(fdt, fdt, fdt),
        mesh=mesh,
        compiler_params=pltpu.CompilerParams(needs_layout_passes=False),
        scratch_types=[
            pltpu.VMEM((n_nodes,), jnp.float32),
            pltpu.VMEM((n_nodes,), jnp.float32),
            pltpu.VMEM((NPAD,), jnp.float32),
            pltpu.VMEM((NPAD,), jnp.float32),
            pltpu.VMEM((NPAD,), jnp.float32),
            pltpu.VMEM((2, e_hi), jnp.int32),
            pltpu.SemaphoreType.DMA,
            pltpu.SemaphoreType.DMA,
            pltpu.SemaphoreType.DMA,
        ],
    )
    def sc_agg(s_hbm, q_hbm, ei_hbm, os_hbm, oq_hbm, od_hbm,
               s_v, q_v, acc_s, acc_q, acc_d, ei_v,
               sem0, sem1, sem2):
        wid = lax.axis_index("c") * NS + lax.axis_index("s")
        is_hi = wid >= nw_lo
        base = jnp.where(is_hi,
                         nw_lo * e_lo + (wid - nw_lo) * e_hi,
                         wid * e_lo)
        cp0 = pltpu.async_copy(s_hbm, s_v, sem0)
        cp1 = pltpu.async_copy(q_hbm, q_v, sem1)
        cp2 = pltpu.async_copy(ei_hbm.at[:, pl.ds(base, e_lo)],
                               ei_v.at[:, pl.ds(0, e_lo)], sem2)

        zeros = jnp.zeros((L,), jnp.float32)

        @plsc.parallel_loop(0, NPAD // L, step=1, unroll=2)
        def zero_body(i):
            b = i * L
            acc_s[pl.ds(b, L)] = zeros
            acc_q[pl.ds(b, L)] = zeros
            acc_d[pl.ds(b, L)] = zeros

        cp0.wait()
        cp1.wait()
        cp2.wait()

        @pl.when(is_hi)
        def _tail_dma():
            pltpu.sync_copy(ei_hbm.at[:, pl.ds(base + e_lo, BLK)],
                            ei_v.at[:, pl.ds(e_lo, BLK)])

        ones = jnp.ones((L,), jnp.float32)

        def _process(b):
            si = ei_v[0, pl.ds(b, L)]
            di = ei_v[1, pl.ds(b, L)]
            sv = plsc.load_gather(s_v, [si])
            qv = plsc.load_gather(q_v, [si])
            plsc.addupdate_scatter(acc_s, [di], sv)
            plsc.addupdate_scatter(acc_q, [di], qv)
            plsc.addupdate_scatter(acc_d, [di], ones)

        @plsc.parallel_loop(0, e_lo, step=L, unroll=4)
        def edge_body(b):
            _process(b)

        @pl.when(is_hi)
        def _tail_edges():
            @plsc.parallel_loop(e_lo, e_hi, step=L, unroll=1)
            def tail_body(b):
                _process(b)

        pltpu.sync_copy(acc_s, os_hbm.at[wid])
        pltpu.sync_copy(acc_q, oq_hbm.at[wid])
        pltpu.sync_copy(acc_d, od_hbm.at[wid])

    return sc_agg


def kernel(h, edge_index, gamma, beta):
    n, d = h.shape
    e = edge_index.shape[1]

    g = pl.cdiv(n, BG)
    s, q = pl.pallas_call(
        _rowsums_body,
        grid=(g,),
        in_specs=[pl.BlockSpec((BG, d), lambda i: (i, 0))],
        out_specs=(
            pl.BlockSpec((BG,), lambda i: (i,)),
            pl.BlockSpec((BG,), lambda i: (i,)),
        ),
        out_shape=(
            jax.ShapeDtypeStruct((n,), jnp.float32),
            jax.ShapeDtypeStruct((n,), jnp.float32),
        ),
    )(h)

    ps, pq, pd = _make_sc_agg(n, e)(s, q, edge_index)

    part_spec = pl.BlockSpec((NW, BG), lambda i: (0, i))
    vec_spec = pl.BlockSpec((d,), lambda i: (0,))
    out = pl.pallas_call(
        _norm_body,
        grid=(g,),
        in_specs=[
            pl.BlockSpec((BG, d), lambda i: (i, 0)),
            part_spec, part_spec, part_spec,
            vec_spec, vec_spec,
        ],
        out_specs=pl.BlockSpec((BG, d), lambda i: (i, 0)),
        out_shape=jax.ShapeDtypeStruct((n, d), h.dtype),
    )(h, ps, pq, pd, gamma, beta)
    return out


# speedup vs baseline: 1.0756x; 1.0007x over previous
import functools

import jax
import jax.numpy as jnp
from jax import lax
from jax.experimental import pallas as pl
from jax.experimental.pallas import tpu as pltpu
from jax.experimental.pallas import tpu_sc as plsc

EPS = 1e-05
NC, NS, L = 2, 16, 16
NW = NC * NS
BG = 5120
NPAD = 10240

BLK = 128


def _rowsums_body(h_ref, s_ref, q_ref):
    hb = h_ref[...]
    s_ref[...] = jnp.sum(hb, axis=1)
    q_ref[...] = jnp.sum(hb * hb, axis=1)


def _norm_body(h_ref, ps_ref, pq_ref, pd_ref, gamma_ref, beta_ref, out_ref):
    hb = h_ref[...]
    d = hb.shape[1]
    ones_m = jnp.ones((8, NW), jnp.float32)
    nb_s = jax.lax.dot(ones_m, ps_ref[...],
                       preferred_element_type=jnp.float32)[0]
    nb_q = jax.lax.dot(ones_m, pq_ref[...],
                       preferred_element_type=jnp.float32)[0]
    deg = jax.lax.dot(ones_m, pd_ref[...],
                      preferred_element_type=jnp.float32)[0]
    total_sum = nb_s + jnp.sum(hb, axis=1)
    total_sq = nb_q + jnp.sum(hb * hb, axis=1)
    n_elem = (deg + 1.0) * d
    mean = total_sum / n_elem
    var = (total_sq - n_elem * mean * mean) / (n_elem - 1.0)
    std = jnp.sqrt(jnp.maximum(var, 0.0))
    has_msg = deg > 0
    mean = jnp.where(has_msg, mean, 0.0)
    std = jnp.where(has_msg, std, 0.0)
    norm_h = (hb - mean[:, None]) / (std[:, None] + EPS)
    out_ref[...] = gamma_ref[...] * norm_h + beta_ref[...]


def _make_sc_agg(n_nodes, n_edges):
    nblk = n_edges // BLK
    blk_lo = nblk // NW
    n_hi = nblk - blk_lo * NW
    nw_lo = NW - n_hi
    e_lo = blk_lo * BLK
    e_hi = e_lo + BLK

    mesh = plsc.VectorSubcoreMesh(core_axis_name="c", subcore_axis_name="s")
    fdt = jax.ShapeDtypeStruct((NW, NPAD), jnp.float32)

    @functools.partial(
        pl.kernel,
        out_type=(fdt, fdt, fdt),
        mesh=mesh,
        compiler_params=pltpu.CompilerParams(needs_layout_passes=False),
        scratch_types=[
            pltpu.VMEM((n_nodes,), jnp.float32),
            pltpu.VMEM((n_nodes,), jnp.float32),
            pltpu.VMEM((NPAD,), jnp.float32),
            pltpu.VMEM((NPAD,), jnp.float32),
            pltpu.VMEM((NPAD,), jnp.float32),
            pltpu.VMEM((2, e_hi), jnp.int32),
            pltpu.SemaphoreType.DMA,
            pltpu.SemaphoreType.DMA,
            pltpu.SemaphoreType.DMA,
        ],
    )
    def sc_agg(s_hbm, q_hbm, ei_hbm, os_hbm, oq_hbm, od_hbm,
               s_v, q_v, acc_s, acc_q, acc_d, ei_v,
               sem0, sem1, sem2):
        wid = lax.axis_index("c") * NS + lax.axis_index("s")
        is_hi = wid >= nw_lo
        base = jnp.where(is_hi,
                         nw_lo * e_lo + (wid - nw_lo) * e_hi,
                         wid * e_lo)
        cp0 = pltpu.async_copy(s_hbm, s_v, sem0)
        cp1 = pltpu.async_copy(q_hbm, q_v, sem1)
        cp2 = pltpu.async_copy(ei_hbm.at[:, pl.ds(base, e_lo)],
                               ei_v.at[:, pl.ds(0, e_lo)], sem2)

        zeros = jnp.zeros((L,), jnp.float32)

        @plsc.parallel_loop(0, NPAD // L, step=1, unroll=2)
        def zero_body(i):
            b = i * L
            acc_s[pl.ds(b, L)] = zeros
            acc_q[pl.ds(b, L)] = zeros
            acc_d[pl.ds(b, L)] = zeros

        cp0.wait()
        cp1.wait()
        cp2.wait()

        @pl.when(is_hi)
        def _tail_dma():
            pltpu.sync_copy(ei_hbm.at[:, pl.ds(base + e_lo, BLK)],
                            ei_v.at[:, pl.ds(e_lo, BLK)])

        ones = jnp.ones((L,), jnp.float32)

        def _process(b):
            si = ei_v[0, pl.ds(b, L)]
            di = ei_v[1, pl.ds(b, L)]
            sv = plsc.load_gather(s_v, [si])
            qv = plsc.load_gather(q_v, [si])
            plsc.addupdate_scatter(acc_s, [di], sv)
            plsc.addupdate_scatter(acc_q, [di], qv)
            plsc.addupdate_scatter(acc_d, [di], ones)

        @plsc.parallel_loop(0, e_lo, step=L, unroll=8)
        def edge_body(b):
            _process(b)

        @pl.when(is_hi)
        def _tail_edges():
            @plsc.parallel_loop(e_lo, e_hi, step=L, unroll=1)
            def tail_body(b):
                _process(b)

        pltpu.sync_copy(acc_s, os_hbm.at[wid])
        pltpu.sync_copy(acc_q, oq_hbm.at[wid])
        pltpu.sync_copy(acc_d, od_hbm.at[wid])

    return sc_agg


def kernel(h, edge_index, gamma, beta):
    n, d = h.shape
    e = edge_index.shape[1]

    g = pl.cdiv(n, BG)
    s, q = pl.pallas_call(
        _rowsums_body,
        grid=(g,),
        in_specs=[pl.BlockSpec((BG, d), lambda i: (i, 0))],
        out_specs=(
            pl.BlockSpec((BG,), lambda i: (i,)),
            pl.BlockSpec((BG,), lambda i: (i,)),
        ),
        out_shape=(
            jax.ShapeDtypeStruct((n,), jnp.float32),
            jax.ShapeDtypeStruct((n,), jnp.float32),
        ),
    )(h)

    ps, pq, pd = _make_sc_agg(n, e)(s, q, edge_index)

    part_spec = pl.BlockSpec((NW, BG), lambda i: (0, i))
    vec_spec = pl.BlockSpec((d,), lambda i: (0,))
    out = pl.pallas_call(
        _norm_body,
        grid=(g,),
        in_specs=[
            pl.BlockSpec((BG, d), lambda i: (i, 0)),
            part_spec, part_spec, part_spec,
            vec_spec, vec_spec,
        ],
        out_specs=pl.BlockSpec((BG, d), lambda i: (i, 0)),
        out_shape=jax.ShapeDtypeStruct((n, d), h.dtype),
    )(h, ps, pq, pd, gamma, beta)
    return out
